# Initial kernel scaffold; baseline (speedup 1.0000x reference)
import jax
import jax.numpy as jnp
from jax import lax
from jax.experimental import pallas as pl
from jax.experimental.pallas import tpu as pltpu
from jax.experimental.pallas import tpu_sc as plsc

_N = 16384
_HALF = _N // 2
_TILES = 16
_RPT = _HALF // _TILES
_ZROWS = 128


def _body(idx0_hbm, idx1_hbm, vals_hbm, out_hbm,
          idx0_v, idx1_v, dst_v, rows_v, zbuf, shared, sem):
    c = lax.axis_index("c")
    s = lax.axis_index("s")
    chunk = idx0_v.shape[0]
    nvec = chunk // 16

    zeros16 = jnp.zeros((16,), jnp.float32)

    def zrow(r, carry):
        zbuf[r, pl.ds(0, 16)] = zeros16
        zbuf[r, pl.ds(16, 16)] = zeros16
        zbuf[r, pl.ds(32, 16)] = zeros16
        zbuf[r, pl.ds(48, 16)] = zeros16
        return carry

    lax.fori_loop(0, _ZROWS, zrow, 0)

    cp0 = pltpu.async_copy(idx0_hbm.at[pl.ds(s * chunk, chunk)], idx0_v, sem)
    cp1 = pltpu.async_copy(idx1_hbm.at[pl.ds(s * chunk, chunk)], idx1_v, sem)

    for k in range(_RPT // _ZROWS):
        pltpu.sync_copy(zbuf, shared.at[pl.ds(s * _RPT + k * _ZROWS, _ZROWS)])

    @pl.when(s == 0)
    def _():
        pltpu.sync_copy(zbuf.at[pl.ds(0, 16)], shared.at[pl.ds(_HALF, 16)])

    cp0.wait()
    cp1.wait()
    plsc.subcore_barrier()

    cbit = c * _HALF
    lanes = lax.iota(jnp.int32, 16)
    base = s * chunk

    def step(j, carry):
        a = idx0_v[pl.ds(j * 16, 16)]
        b = idx1_v[pl.ds(j * 16, 16)]
        act = (a == b) & ((a & _HALF) == cbit)

        @pl.when(jnp.any(act))
        def _():
            gi = base + j * 16 + lanes
            src = jnp.where(act, gi, 0)
            dst_v[...] = jnp.where(act, a & (_HALF - 1), _HALF)
            pltpu.async_copy(vals_hbm.at[src], rows_v, sem).wait()
            pltpu.sync_copy(rows_v, shared.at[dst_v], add=True)

        return carry

    lax.fori_loop(0, nvec, step, 0)
    plsc.subcore_barrier()

    pltpu.sync_copy(shared.at[pl.ds(s * _RPT, _RPT)],
                    out_hbm.at[pl.ds(c * _HALF + s * _RPT, _RPT)])


def kernel(indices, values):
    nnz, d = values.shape
    chunk = ((nnz + 16 * _TILES - 1) // (16 * _TILES)) * 16
    pad = _TILES * chunk - nnz
    idx0 = jnp.concatenate([indices[0], jnp.zeros((pad,), jnp.int32)])
    idx1 = jnp.concatenate([indices[1], jnp.ones((pad,), jnp.int32)])

    mesh = plsc.VectorSubcoreMesh(core_axis_name="c", subcore_axis_name="s")
    f = pl.kernel(
        _body,
        mesh=mesh,
        out_type=jax.ShapeDtypeStruct((_N, d), jnp.float32),
        scratch_types=[
            pltpu.VMEM((chunk,), jnp.int32),
            pltpu.VMEM((chunk,), jnp.int32),
            pltpu.VMEM((16,), jnp.int32),
            pltpu.VMEM((16, d), jnp.float32),
            pltpu.VMEM((_ZROWS, d), jnp.float32),
            pltpu.VMEM_SHARED((_HALF + 16, d), jnp.float32),
            pltpu.SemaphoreType.DMA,
        ],
    )
    return f(idx0, idx1, values)


# speedup vs baseline: 7.0860x; 7.0860x over previous
import jax
import jax.numpy as jnp
from jax import lax
from jax.experimental import pallas as pl
from jax.experimental.pallas import tpu as pltpu
from jax.experimental.pallas import tpu_sc as plsc

_N = 16384
_HALF = _N // 2
_TILES = 16
_RPT = _HALF // _TILES
_ZROWS = 128


def _body(idx0_hbm, idx1_hbm, vals_hbm, out_hbm,
          idx0_v, idx1_v, dst_v, rows_v, zbuf, shared, sem):
    c = lax.axis_index("c")
    s = lax.axis_index("s")
    chunk = idx0_v.shape[0]
    nvec = chunk // 16

    zeros16 = jnp.zeros((16,), jnp.float32)

    def zrow(r, carry):
        zbuf[r, pl.ds(0, 16)] = zeros16
        zbuf[r, pl.ds(16, 16)] = zeros16
        zbuf[r, pl.ds(32, 16)] = zeros16
        zbuf[r, pl.ds(48, 16)] = zeros16
        return carry

    lax.fori_loop(0, _ZROWS, zrow, 0)

    cp0 = pltpu.async_copy(idx0_hbm.at[pl.ds(s * chunk, chunk)], idx0_v, sem)
    cp1 = pltpu.async_copy(idx1_hbm.at[pl.ds(s * chunk, chunk)], idx1_v, sem)

    for k in range(_RPT // _ZROWS):
        pltpu.sync_copy(zbuf, shared.at[pl.ds(s * _RPT + k * _ZROWS, _ZROWS)])

    @pl.when(s == 0)
    def _():
        pltpu.sync_copy(zbuf.at[pl.ds(0, 16)], shared.at[pl.ds(_HALF, 16)])

    cp0.wait()
    cp1.wait()
    plsc.subcore_barrier()

    cbit = c * _HALF
    lanes = lax.iota(jnp.int32, 16)
    base = s * chunk

    def step(j, carry):
        a = idx0_v[pl.ds(j * 16, 16)]
        b = idx1_v[pl.ds(j * 16, 16)]
        act = (a == b) & ((a & _HALF) == cbit)
        nhit = plsc.all_reduce_population_count(act)

        @pl.when(nhit[0] > 0)
        def _():
            gi = base + j * 16 + lanes
            src = jnp.where(act, gi, 0)
            dst_v[...] = jnp.where(act, a & (_HALF - 1), _HALF)
            pltpu.async_copy(vals_hbm.at[src], rows_v, sem).wait()
            pltpu.sync_copy(rows_v, shared.at[dst_v], add=True)

        return carry

    lax.fori_loop(0, nvec, step, 0)
    plsc.subcore_barrier()

    pltpu.sync_copy(shared.at[pl.ds(s * _RPT, _RPT)],
                    out_hbm.at[pl.ds(c * _HALF + s * _RPT, _RPT)])


def kernel(indices, values):
    nnz, d = values.shape
    chunk = ((nnz + 16 * _TILES - 1) // (16 * _TILES)) * 16
    pad = _TILES * chunk - nnz
    idx0 = jnp.concatenate([indices[0], jnp.zeros((pad,), jnp.int32)])
    idx1 = jnp.concatenate([indices[1], jnp.ones((pad,), jnp.int32)])

    mesh = plsc.VectorSubcoreMesh(core_axis_name="c", subcore_axis_name="s")
    f = pl.kernel(
        _body,
        mesh=mesh,
        out_type=jax.ShapeDtypeStruct((_N, d), jnp.float32),
        scratch_types=[
            pltpu.VMEM((chunk,), jnp.int32),
            pltpu.VMEM((chunk,), jnp.int32),
            pltpu.VMEM((16,), jnp.int32),
            pltpu.VMEM((16, d), jnp.float32),
            pltpu.VMEM((_ZROWS, d), jnp.float32),
            pltpu.VMEM_SHARED((_HALF + 16, d), jnp.float32),
            pltpu.SemaphoreType.DMA,
        ],
        compiler_params=pltpu.CompilerParams(
            needs_layout_passes=False, use_tc_tiling_on_sc=False),
    )
    return f(idx0, idx1, values)


# speedup vs baseline: 7.1375x; 1.0073x over previous
import jax
import jax.numpy as jnp
from jax import lax
from jax.experimental import pallas as pl
from jax.experimental.pallas import tpu as pltpu
from jax.experimental.pallas import tpu_sc as plsc

_N = 16384
_D = 64
_HALF = _N // 2
_TILES = 16
_RPT = _HALF // _TILES
_ZROWS = 128


def _body(idx0_hbm, idx1_hbm, vals_hbm, out_hbm,
          idx0_v, idx1_v, dst_v, rows_v, zbuf, shared, sem):
    c = lax.axis_index("c")
    s = lax.axis_index("s")
    chunk = idx0_v.shape[0]
    nvec = chunk // 16

    zeros16 = jnp.zeros((16,), jnp.float32)

    def zrow(r, carry):
        zbuf[r, pl.ds(0, 16)] = zeros16
        zbuf[r, pl.ds(16, 16)] = zeros16
        zbuf[r, pl.ds(32, 16)] = zeros16
        zbuf[r, pl.ds(48, 16)] = zeros16
        return carry

    lax.fori_loop(0, _ZROWS, zrow, 0)

    cp0 = pltpu.async_copy(idx0_hbm.at[pl.ds(s * chunk, chunk)], idx0_v, sem)
    cp1 = pltpu.async_copy(idx1_hbm.at[pl.ds(s * chunk, chunk)], idx1_v, sem)

    for k in range(_RPT // _ZROWS):
        pltpu.sync_copy(zbuf, shared.at[pl.ds(s * _RPT + k * _ZROWS, _ZROWS)])

    @pl.when(s == 0)
    def _():
        pltpu.sync_copy(zbuf.at[pl.ds(0, 16)], shared.at[pl.ds(_HALF, 16)])

    cp0.wait()
    cp1.wait()
    plsc.subcore_barrier()

    cbit = c * _HALF
    lanes = lax.iota(jnp.int32, 16)
    base = s * chunk

    def step(j, carry):
        a = idx0_v[pl.ds(j * 16, 16)]
        b = idx1_v[pl.ds(j * 16, 16)]
        act = (a == b) & ((a & _HALF) == cbit)
        nhit = plsc.all_reduce_population_count(act)

        @pl.when(nhit[0] > 0)
        def _():
            gi = base + j * 16 + lanes
            src = jnp.where(act, gi, 0)
            dst_v[...] = jnp.where(act, a & (_HALF - 1), _HALF)
            handles = [
                pltpu.async_copy(
                    vals_hbm.at[pl.ds(src[l] * _D, _D)], rows_v.at[l], sem)
                for l in range(16)
            ]
            for h in handles:
                h.wait()
            pltpu.sync_copy(rows_v, shared.at[dst_v], add=True)

        return carry

    lax.fori_loop(0, nvec, step, 0)
    plsc.subcore_barrier()

    pltpu.sync_copy(shared.at[pl.ds(s * _RPT, _RPT)],
                    out_hbm.at[pl.ds(c * _HALF + s * _RPT, _RPT)])


def kernel(indices, values):
    nnz, d = values.shape
    chunk = ((nnz + 16 * _TILES - 1) // (16 * _TILES)) * 16
    pad = _TILES * chunk - nnz
    idx0 = jnp.concatenate([indices[0], jnp.zeros((pad,), jnp.int32)])
    idx1 = jnp.concatenate([indices[1], jnp.ones((pad,), jnp.int32)])

    mesh = plsc.VectorSubcoreMesh(core_axis_name="c", subcore_axis_name="s")
    f = pl.kernel(
        _body,
        mesh=mesh,
        out_type=jax.ShapeDtypeStruct((_N, d), jnp.float32),
        scratch_types=[
            pltpu.VMEM((chunk,), jnp.int32),
            pltpu.VMEM((chunk,), jnp.int32),
            pltpu.VMEM((16,), jnp.int32),
            pltpu.VMEM((16, d), jnp.float32),
            pltpu.VMEM((_ZROWS, d), jnp.float32),
            pltpu.VMEM_SHARED((_HALF + 16, d), jnp.float32),
            pltpu.SemaphoreType.DMA,
        ],
        compiler_params=pltpu.CompilerParams(
            needs_layout_passes=False, use_tc_tiling_on_sc=False),
    )
    return f(idx0, idx1, values.reshape(-1))


# speedup vs baseline: 9.5246x; 1.3344x over previous
import jax
import jax.numpy as jnp
from jax import lax
from jax.experimental import pallas as pl
from jax.experimental.pallas import tpu as pltpu
from jax.experimental.pallas import tpu_sc as plsc

_N = 16384
_D = 64
_W = 128
_HALF = _N // 2
_TILES = 16
_RPT = _HALF // _TILES
_ZROWS = 64


def _body(idx0_hbm, idx1_hbm, vals_hbm, out_hbm,
          idx0_v, idx1_v, dst_v, rows_v, zbuf, shared, sem):
    c = lax.axis_index("c")
    s = lax.axis_index("s")
    chunk = idx0_v.shape[0]
    nvec = chunk // 16

    zeros16 = jnp.zeros((16,), jnp.float32)

    def zrow(r, carry):
        for col in range(0, _W, 16):
            zbuf[r, pl.ds(col, 16)] = zeros16
        return carry

    lax.fori_loop(0, _ZROWS, zrow, 0)

    cp0 = pltpu.async_copy(idx0_hbm.at[pl.ds(s * chunk, chunk)], idx0_v, sem)
    cp1 = pltpu.async_copy(idx1_hbm.at[pl.ds(s * chunk, chunk)], idx1_v, sem)

    for k in range(_RPT // _ZROWS):
        pltpu.sync_copy(zbuf, shared.at[pl.ds(s * _RPT + k * _ZROWS, _ZROWS)])

    @pl.when(s == 0)
    def _():
        pltpu.sync_copy(zbuf.at[pl.ds(0, 16)], shared.at[pl.ds(_HALF, 16)])

    cp0.wait()
    cp1.wait()
    plsc.subcore_barrier()

    cbit = c * _HALF
    lanes = lax.iota(jnp.int32, 16)
    base = s * chunk

    def step(j, carry):
        a = idx0_v[pl.ds(j * 16, 16)]
        b = idx1_v[pl.ds(j * 16, 16)]
        act = (a == b) & ((a & _HALF) == cbit)
        nhit = plsc.all_reduce_population_count(act)

        @pl.when(nhit[0] > 0)
        def _():
            gi = base + j * 16 + lanes
            src = jnp.where(act, gi, 0)
            dst_v[...] = jnp.where(act, a & (_HALF - 1), _HALF)
            handles = [
                pltpu.async_copy(
                    vals_hbm.at[src[l]], rows_v.at[l, pl.ds(0, _D)], sem)
                for l in range(16)
            ]
            for h in handles:
                h.wait()
            pltpu.sync_copy(rows_v, shared.at[dst_v], add=True)

        return carry

    lax.fori_loop(0, nvec, step, 0)
    plsc.subcore_barrier()

    pltpu.sync_copy(shared.at[pl.ds(s * _RPT, _RPT)],
                    out_hbm.at[pl.ds(c * _HALF + s * _RPT, _RPT)])


def kernel(indices, values):
    nnz, d = values.shape
    chunk = ((nnz + 16 * _TILES - 1) // (16 * _TILES)) * 16
    pad = _TILES * chunk - nnz
    idx0 = jnp.concatenate([indices[0], jnp.zeros((pad,), jnp.int32)])
    idx1 = jnp.concatenate([indices[1], jnp.ones((pad,), jnp.int32)])

    mesh = plsc.VectorSubcoreMesh(core_axis_name="c", subcore_axis_name="s")
    f = pl.kernel(
        _body,
        mesh=mesh,
        out_type=jax.ShapeDtypeStruct((_N, _W), jnp.float32),
        scratch_types=[
            pltpu.VMEM((chunk,), jnp.int32),
            pltpu.VMEM((chunk,), jnp.int32),
            pltpu.VMEM((16,), jnp.int32),
            pltpu.VMEM((16, _W), jnp.float32),
            pltpu.VMEM((_ZROWS, _W), jnp.float32),
            pltpu.VMEM_SHARED((_HALF + 16, _W), jnp.float32),
            pltpu.SemaphoreType.DMA,
        ],
        compiler_params=pltpu.CompilerParams(needs_layout_passes=False),
    )
    return f(idx0, idx1, values)[:, :d]


# speedup vs baseline: 10.6319x; 1.1163x over previous
import jax
import jax.numpy as jnp
from jax import lax
from jax.experimental import pallas as pl
from jax.experimental.pallas import tpu as pltpu
from jax.experimental.pallas import tpu_sc as plsc

_N = 16384
_D = 64
_W = 128
_HALF = _N // 2
_TILES = 16
_RPT = _HALF // _TILES
_ZROWS = 64
_UNROLL = 8


def _body(idx0_hbm, idx1_hbm, vals_hbm, out_hbm,
          idx0_v, idx1_v, dst_v, rows_v, zbuf, shared, sem):
    c = lax.axis_index("c")
    s = lax.axis_index("s")
    chunk = idx0_v.shape[0]
    nvec = chunk // 16

    zeros16 = jnp.zeros((16,), jnp.float32)

    def zrow(r, carry):
        for col in range(0, _W, 16):
            zbuf[r, pl.ds(col, 16)] = zeros16
        return carry

    lax.fori_loop(0, _ZROWS, zrow, 0)

    cp0 = pltpu.async_copy(idx0_hbm.at[pl.ds(s * chunk, chunk)], idx0_v, sem)
    cp1 = pltpu.async_copy(idx1_hbm.at[pl.ds(s * chunk, chunk)], idx1_v, sem)

    for k in range(_RPT // _ZROWS):
        pltpu.sync_copy(zbuf, shared.at[pl.ds(s * _RPT + k * _ZROWS, _ZROWS)])

    @pl.when(s == 0)
    def _():
        pltpu.sync_copy(zbuf.at[pl.ds(0, 16)], shared.at[pl.ds(_HALF, 16)])

    cp0.wait()
    cp1.wait()
    plsc.subcore_barrier()

    cbit = c * _HALF
    lanes = lax.iota(jnp.int32, 16)
    base = s * chunk

    def step(j, carry):
        acts = []
        avs = []
        for u in range(_UNROLL):
            a = idx0_v[pl.ds((j * _UNROLL + u) * 16, 16)]
            b = idx1_v[pl.ds((j * _UNROLL + u) * 16, 16)]
            avs.append(a)
            acts.append((a == b) & ((a & _HALF) == cbit))
        blk = acts[0]
        for u in range(1, _UNROLL):
            blk = blk | acts[u]
        nhit = plsc.all_reduce_population_count(blk)

        @pl.when(nhit[0] > 0)
        def _():
            for u in range(_UNROLL):
                act = acts[u]
                a = avs[u]
                nh = plsc.all_reduce_population_count(act)

                @pl.when(nh[0] > 0)
                def _(u=u, act=act, a=a):
                    gi = base + (j * _UNROLL + u) * 16 + lanes
                    src = jnp.where(act, gi, 0)
                    dst_v[...] = jnp.where(act, a & (_HALF - 1), _HALF)
                    handles = [
                        pltpu.async_copy(
                            vals_hbm.at[src[l]],
                            rows_v.at[l, pl.ds(0, _D)], sem)
                        for l in range(16)
                    ]
                    for h in handles:
                        h.wait()
                    pltpu.sync_copy(rows_v, shared.at[dst_v], add=True)

        return carry

    lax.fori_loop(0, nvec // _UNROLL, step, 0)
    plsc.subcore_barrier()

    pltpu.sync_copy(shared.at[pl.ds(s * _RPT, _RPT)],
                    out_hbm.at[pl.ds(c * _HALF + s * _RPT, _RPT)])


def kernel(indices, values):
    nnz, d = values.shape
    grain = 16 * _UNROLL
    chunk = ((nnz + grain * _TILES - 1) // (grain * _TILES)) * grain
    pad = _TILES * chunk - nnz
    idx0 = jnp.concatenate([indices[0], jnp.zeros((pad,), jnp.int32)])
    idx1 = jnp.concatenate([indices[1], jnp.ones((pad,), jnp.int32)])

    mesh = plsc.VectorSubcoreMesh(core_axis_name="c", subcore_axis_name="s")
    f = pl.kernel(
        _body,
        mesh=mesh,
        out_type=jax.ShapeDtypeStruct((_N, _W), jnp.float32),
        scratch_types=[
            pltpu.VMEM((chunk,), jnp.int32),
            pltpu.VMEM((chunk,), jnp.int32),
            pltpu.VMEM((16,), jnp.int32),
            pltpu.VMEM((16, _W), jnp.float32),
            pltpu.VMEM((_ZROWS, _W), jnp.float32),
            pltpu.VMEM_SHARED((_HALF + 16, _W), jnp.float32),
            pltpu.SemaphoreType.DMA,
        ],
        compiler_params=pltpu.CompilerParams(needs_layout_passes=False),
    )
    return f(idx0, idx1, values)[:, :d]
